# baseline (speedup 1.0000x reference)
import functools

import jax
import jax.numpy as jnp
from jax.experimental import pallas as pl
from jax.experimental.pallas import tpu as pltpu

HIDDEN = 768
FF = 3072
E = 8
TOP_K = 2
T = 2048

BLK = 256
G = (T * TOP_K) // BLK + E


def _router_body(x_ref, wg_ref, bg_ref, idx_ref, gates_ref):
    x = x_ref[...]
    logits = jax.lax.dot_general(
        x, wg_ref[...], (((1,), (0,)), ((), ())),
        preferred_element_type=jnp.float32) + bg_ref[...][None, :]
    col = jax.lax.broadcasted_iota(jnp.int32, (T, E), 1)
    m1 = jnp.max(logits, axis=1, keepdims=True)
    i1 = jnp.min(jnp.where(logits == m1, col, E), axis=1, keepdims=True)
    masked = jnp.where(col == i1, -jnp.inf, logits)
    m2 = jnp.max(masked, axis=1, keepdims=True)
    i2 = jnp.min(jnp.where(masked == m2, col, E), axis=1, keepdims=True)
    e2 = jnp.exp(m2 - m1)
    g1 = 1.0 / (1.0 + e2)
    g2 = e2 * g1
    idx_ref[...] = jnp.concatenate([i1, i2], axis=1)
    gates_ref[...] = jnp.concatenate([g1, g2], axis=1)


def _moe_body(be_ref, na_ref, x_ref, d0_ref, d1_ref, gw0_ref, gw1_ref,
              w1_ref, b1_ref, w2_ref, b2_ref, out_ref):
    g = pl.program_id(0)

    @pl.when(g == 0)
    def _():
        out_ref[...] = jnp.zeros_like(out_ref)

    @pl.when(g < na_ref[0])
    def _():
        row_ids = g * BLK + jax.lax.broadcasted_iota(jnp.int32, (BLK, T), 0)
        cmp0 = d0_ref[...] == row_ids
        cmp1 = d1_ref[...] == row_ids
        gmask = jnp.where(cmp0 | cmp1, 1.0, 0.0).astype(jnp.bfloat16)
        rows = jax.lax.dot_general(
            gmask, x_ref[...], (((1,), (0,)), ((), ())),
            preferred_element_type=jnp.float32).astype(jnp.bfloat16)
        h = jax.nn.gelu(jax.lax.dot_general(
            rows, w1_ref[0], (((1,), (0,)), ((), ())),
            preferred_element_type=jnp.float32) + b1_ref[0])
        eo = (jax.lax.dot_general(
            h.astype(jnp.bfloat16), w2_ref[0], (((1,), (0,)), ((), ())),
            preferred_element_type=jnp.float32) + b2_ref[0]).astype(jnp.bfloat16)
        gw = (jnp.where(cmp0, gw0_ref[...], 0.0)
              + jnp.where(cmp1, gw1_ref[...], 0.0)).astype(jnp.bfloat16)
        out_ref[...] += jax.lax.dot_general(
            gw, eo, (((0,), (0,)), ((), ())),
            preferred_element_type=jnp.float32)


def kernel(x, Wg, bg, W1, b1, W2, b2):
    B, S, H = x.shape
    x_flat = x.reshape(-1, H)

    top_idx, gates = pl.pallas_call(
        _router_body,
        grid=(1,),
        in_specs=[
            pl.BlockSpec((T, HIDDEN), lambda i: (0, 0)),
            pl.BlockSpec((HIDDEN, E), lambda i: (0, 0)),
            pl.BlockSpec((E,), lambda i: (0,)),
        ],
        out_specs=[
            pl.BlockSpec((T, TOP_K), lambda i: (0, 0)),
            pl.BlockSpec((T, TOP_K), lambda i: (0, 0)),
        ],
        out_shape=[
            jax.ShapeDtypeStruct((T, TOP_K), jnp.int32),
            jax.ShapeDtypeStruct((T, TOP_K), jnp.float32),
        ],
    )(x_flat, Wg, bg)

    oh = top_idx[..., None] == jnp.arange(E)[None, None, :]
    c = oh.sum(1).astype(jnp.int32)
    incl = jnp.cumsum(c, axis=0)
    excl = incl - c
    counts = incl[-1]
    blocks_e = (counts + BLK - 1) // BLK
    cumB = jnp.cumsum(blocks_e)
    row_start = (cumB - blocks_e) * BLK
    base = row_start[None, :] + excl
    dest0 = jnp.sum(jnp.where(oh[:, 0], base, 0), axis=-1).astype(jnp.int32)
    dest1 = jnp.sum(jnp.where(oh[:, 1], base, 0), axis=-1).astype(jnp.int32)
    n_active = cumB[-1]
    g_idx = jnp.arange(G, dtype=jnp.int32)
    be = jnp.clip((g_idx[:, None] >= cumB[None, :]).sum(-1), 0, E - 1)
    be = jnp.where(g_idx < n_active, be, be[n_active - 1]).astype(jnp.int32)

    out = pl.pallas_call(
        _moe_body,
        grid_spec=pltpu.PrefetchScalarGridSpec(
            num_scalar_prefetch=2,
            grid=(G,),
            in_specs=[
                pl.BlockSpec((T, HIDDEN), lambda g, be_r, na_r: (0, 0)),
                pl.BlockSpec((1, T), lambda g, be_r, na_r: (0, 0)),
                pl.BlockSpec((1, T), lambda g, be_r, na_r: (0, 0)),
                pl.BlockSpec((1, T), lambda g, be_r, na_r: (0, 0)),
                pl.BlockSpec((1, T), lambda g, be_r, na_r: (0, 0)),
                pl.BlockSpec((1, HIDDEN, FF),
                             lambda g, be_r, na_r: (be_r[g], 0, 0)),
                pl.BlockSpec((1, 1, FF),
                             lambda g, be_r, na_r: (be_r[g], 0, 0)),
                pl.BlockSpec((1, FF, HIDDEN),
                             lambda g, be_r, na_r: (be_r[g], 0, 0)),
                pl.BlockSpec((1, 1, HIDDEN),
                             lambda g, be_r, na_r: (be_r[g], 0, 0)),
            ],
            out_specs=pl.BlockSpec((T, HIDDEN), lambda g, be_r, na_r: (0, 0)),
        ),
        out_shape=jax.ShapeDtypeStruct((T, HIDDEN), jnp.float32),
    )(be, jnp.reshape(n_active, (1,)), x_flat.astype(jnp.bfloat16),
      dest0.reshape(1, T), dest1.reshape(1, T),
      gates[:, 0].reshape(1, T), gates[:, 1].reshape(1, T),
      W1.astype(jnp.bfloat16), b1.reshape(E, 1, FF),
      W2.astype(jnp.bfloat16), b2.reshape(E, 1, HIDDEN))
    return out.reshape(B, S, H)


# speedup vs baseline: 1.0732x; 1.0732x over previous
import functools

import jax
import jax.numpy as jnp
from jax.experimental import pallas as pl
from jax.experimental.pallas import tpu as pltpu

HIDDEN = 768
FF = 3072
E = 8
TOP_K = 2
T = 2048

BLK = 256
G = (T * TOP_K) // BLK + E
FB = 1536
NF = FF // FB


def _router_body(x_ref, wg_ref, bg_ref, idx_ref, gates_ref):
    x = x_ref[...]
    logits = jax.lax.dot_general(
        x, wg_ref[...], (((1,), (0,)), ((), ())),
        preferred_element_type=jnp.float32) + bg_ref[...][None, :]
    col = jax.lax.broadcasted_iota(jnp.int32, (T, E), 1)
    m1 = jnp.max(logits, axis=1, keepdims=True)
    i1 = jnp.min(jnp.where(logits == m1, col, E), axis=1, keepdims=True)
    masked = jnp.where(col == i1, -jnp.inf, logits)
    m2 = jnp.max(masked, axis=1, keepdims=True)
    i2 = jnp.min(jnp.where(masked == m2, col, E), axis=1, keepdims=True)
    e2 = jnp.exp(m2 - m1)
    g1 = 1.0 / (1.0 + e2)
    g2 = e2 * g1
    idx_ref[...] = jnp.concatenate([i1, i2], axis=1)
    gates_ref[...] = jnp.concatenate([g1, g2], axis=1)


def _moe_body(be_ref, na_ref, x_ref, d0_ref, d1_ref, gw0_ref, gw1_ref,
              w1_ref, b1_ref, w2_ref, b2_ref, out_ref, rows_s, acc_s):
    g = pl.program_id(0)
    f = pl.program_id(1)

    @pl.when((g == 0) & (f == 0))
    def _():
        out_ref[...] = jnp.zeros_like(out_ref)

    @pl.when(g < na_ref[0])
    def _():
        row_ids = g * BLK + jax.lax.broadcasted_iota(jnp.int32, (BLK, T), 0)
        cmp0 = d0_ref[...] == row_ids
        cmp1 = d1_ref[...] == row_ids

        @pl.when(f == 0)
        def _():
            gmask = jnp.where(cmp0 | cmp1, 1.0, 0.0).astype(jnp.bfloat16)
            rows_s[...] = jax.lax.dot_general(
                gmask, x_ref[...], (((1,), (0,)), ((), ())),
                preferred_element_type=jnp.float32).astype(jnp.bfloat16)

        h = jax.nn.gelu(jax.lax.dot_general(
            rows_s[...], w1_ref[0].astype(jnp.bfloat16),
            (((1,), (0,)), ((), ())),
            preferred_element_type=jnp.float32) + b1_ref[0])
        contrib = jax.lax.dot_general(
            h.astype(jnp.bfloat16), w2_ref[0].astype(jnp.bfloat16),
            (((1,), (0,)), ((), ())),
            preferred_element_type=jnp.float32)

        @pl.when(f == 0)
        def _():
            acc_s[...] = contrib

        @pl.when(f == NF - 1)
        def _():
            eo = (acc_s[...] + contrib + b2_ref[0]).astype(jnp.bfloat16)
            gw = (jnp.where(cmp0, gw0_ref[...], 0.0)
                  + jnp.where(cmp1, gw1_ref[...], 0.0)).astype(jnp.bfloat16)
            out_ref[...] += jax.lax.dot_general(
                gw, eo, (((0,), (0,)), ((), ())),
                preferred_element_type=jnp.float32)


def kernel(x, Wg, bg, W1, b1, W2, b2):
    B, S, H = x.shape
    x_flat = x.reshape(-1, H)

    top_idx, gates = pl.pallas_call(
        _router_body,
        grid=(1,),
        in_specs=[
            pl.BlockSpec((T, HIDDEN), lambda i: (0, 0)),
            pl.BlockSpec((HIDDEN, E), lambda i: (0, 0)),
            pl.BlockSpec((E,), lambda i: (0,)),
        ],
        out_specs=[
            pl.BlockSpec((T, TOP_K), lambda i: (0, 0)),
            pl.BlockSpec((T, TOP_K), lambda i: (0, 0)),
        ],
        out_shape=[
            jax.ShapeDtypeStruct((T, TOP_K), jnp.int32),
            jax.ShapeDtypeStruct((T, TOP_K), jnp.float32),
        ],
    )(x_flat, Wg, bg)

    oh = top_idx[..., None] == jnp.arange(E)[None, None, :]
    c = oh.sum(1).astype(jnp.int32)
    incl = jnp.cumsum(c, axis=0)
    excl = incl - c
    counts = incl[-1]
    blocks_e = (counts + BLK - 1) // BLK
    cumB = jnp.cumsum(blocks_e)
    row_start = (cumB - blocks_e) * BLK
    base = row_start[None, :] + excl
    dest0 = jnp.sum(jnp.where(oh[:, 0], base, 0), axis=-1).astype(jnp.int32)
    dest1 = jnp.sum(jnp.where(oh[:, 1], base, 0), axis=-1).astype(jnp.int32)
    n_active = cumB[-1]
    g_idx = jnp.arange(G, dtype=jnp.int32)
    be = jnp.clip((g_idx[:, None] >= cumB[None, :]).sum(-1), 0, E - 1)
    be = jnp.where(g_idx < n_active, be, be[n_active - 1]).astype(jnp.int32)

    out = pl.pallas_call(
        _moe_body,
        grid_spec=pltpu.PrefetchScalarGridSpec(
            num_scalar_prefetch=2,
            grid=(G, NF),
            in_specs=[
                pl.BlockSpec((T, HIDDEN), lambda g, f, be_r, na_r: (0, 0)),
                pl.BlockSpec((1, T), lambda g, f, be_r, na_r: (0, 0)),
                pl.BlockSpec((1, T), lambda g, f, be_r, na_r: (0, 0)),
                pl.BlockSpec((1, T), lambda g, f, be_r, na_r: (0, 0)),
                pl.BlockSpec((1, T), lambda g, f, be_r, na_r: (0, 0)),
                pl.BlockSpec((1, HIDDEN, FB),
                             lambda g, f, be_r, na_r: (be_r[g], 0, f)),
                pl.BlockSpec((1, 1, FB),
                             lambda g, f, be_r, na_r: (be_r[g], 0, f)),
                pl.BlockSpec((1, FB, HIDDEN),
                             lambda g, f, be_r, na_r: (be_r[g], f, 0)),
                pl.BlockSpec((1, 1, HIDDEN),
                             lambda g, f, be_r, na_r: (be_r[g], 0, 0)),
            ],
            out_specs=pl.BlockSpec((T, HIDDEN),
                                   lambda g, f, be_r, na_r: (0, 0)),
            scratch_shapes=[
                pltpu.VMEM((BLK, HIDDEN), jnp.bfloat16),
                pltpu.VMEM((BLK, HIDDEN), jnp.float32),
            ],
        ),
        out_shape=jax.ShapeDtypeStruct((T, HIDDEN), jnp.float32),
    )(be, jnp.reshape(n_active, (1,)), x_flat.astype(jnp.bfloat16),
      dest0.reshape(1, T), dest1.reshape(1, T),
      gates[:, 0].reshape(1, T), gates[:, 1].reshape(1, T),
      W1, b1.reshape(E, 1, FF), W2, b2.reshape(E, 1, HIDDEN))
    return out.reshape(B, S, H)


# speedup vs baseline: 1.4207x; 1.3238x over previous
import functools

import jax
import jax.numpy as jnp
from jax.experimental import pallas as pl
from jax.experimental.pallas import tpu as pltpu

HIDDEN = 768
FF = 3072
E = 8
TOP_K = 2
T = 2048

BLK = 256
G = (T * TOP_K) // BLK + E


def _router_body(x_ref, wg_ref, bg_ref, idx_ref, gates_ref):
    x = x_ref[...]
    logits = jax.lax.dot_general(
        x, wg_ref[...], (((1,), (0,)), ((), ())),
        preferred_element_type=jnp.float32) + bg_ref[...][None, :]
    col = jax.lax.broadcasted_iota(jnp.int32, (T, E), 1)
    m1 = jnp.max(logits, axis=1, keepdims=True)
    i1 = jnp.min(jnp.where(logits == m1, col, E), axis=1, keepdims=True)
    masked = jnp.where(col == i1, -jnp.inf, logits)
    m2 = jnp.max(masked, axis=1, keepdims=True)
    i2 = jnp.min(jnp.where(masked == m2, col, E), axis=1, keepdims=True)
    e2 = jnp.exp(m2 - m1)
    g1 = 1.0 / (1.0 + e2)
    g2 = e2 * g1
    idx_ref[...] = jnp.concatenate([i1, i2], axis=1)
    gates_ref[...] = jnp.concatenate([g1, g2], axis=1)


def _moe_body(be_ref, na_ref, x_ref, d0_ref, d1_ref, gw0_ref, gw1_ref,
              w1_ref, b1_ref, w2_ref, b2_ref, out_ref):
    g = pl.program_id(0)

    @pl.when(g == 0)
    def _():
        out_ref[...] = jnp.zeros_like(out_ref)

    @pl.when(g < na_ref[0])
    def _():
        row_ids = g * BLK + jax.lax.broadcasted_iota(jnp.int32, (BLK, T), 0)
        cmp0 = d0_ref[...] == row_ids
        cmp1 = d1_ref[...] == row_ids
        gmask = jnp.where(cmp0 | cmp1, 1.0, 0.0)
        rows = jax.lax.dot_general(
            gmask, x_ref[...], (((1,), (0,)), ((), ())),
            preferred_element_type=jnp.float32)
        h = jax.nn.gelu(jax.lax.dot_general(
            rows, w1_ref[0], (((1,), (0,)), ((), ())),
            preferred_element_type=jnp.float32) + b1_ref[0])
        eo = jax.lax.dot_general(
            h, w2_ref[0], (((1,), (0,)), ((), ())),
            preferred_element_type=jnp.float32) + b2_ref[0]
        gw = (jnp.where(cmp0, gw0_ref[...], 0.0)
              + jnp.where(cmp1, gw1_ref[...], 0.0))
        out_ref[...] += jax.lax.dot_general(
            gw, eo, (((0,), (0,)), ((), ())),
            preferred_element_type=jnp.float32)


def kernel(x, Wg, bg, W1, b1, W2, b2):
    B, S, H = x.shape
    x_flat = x.reshape(-1, H)

    top_idx, gates = pl.pallas_call(
        _router_body,
        grid=(1,),
        in_specs=[
            pl.BlockSpec((T, HIDDEN), lambda i: (0, 0)),
            pl.BlockSpec((HIDDEN, E), lambda i: (0, 0)),
            pl.BlockSpec((E,), lambda i: (0,)),
        ],
        out_specs=[
            pl.BlockSpec((T, TOP_K), lambda i: (0, 0)),
            pl.BlockSpec((T, TOP_K), lambda i: (0, 0)),
        ],
        out_shape=[
            jax.ShapeDtypeStruct((T, TOP_K), jnp.int32),
            jax.ShapeDtypeStruct((T, TOP_K), jnp.float32),
        ],
    )(x_flat, Wg, bg)

    oh = top_idx[..., None] == jnp.arange(E)[None, None, :]
    c = oh.sum(1).astype(jnp.int32)
    incl = jnp.cumsum(c, axis=0)
    excl = incl - c
    counts = incl[-1]
    blocks_e = (counts + BLK - 1) // BLK
    cumB = jnp.cumsum(blocks_e)
    row_start = (cumB - blocks_e) * BLK
    base = row_start[None, :] + excl
    dest0 = jnp.sum(jnp.where(oh[:, 0], base, 0), axis=-1).astype(jnp.int32)
    dest1 = jnp.sum(jnp.where(oh[:, 1], base, 0), axis=-1).astype(jnp.int32)
    n_active = cumB[-1]
    g_idx = jnp.arange(G, dtype=jnp.int32)
    be = jnp.clip((g_idx[:, None] >= cumB[None, :]).sum(-1), 0, E - 1)
    be = jnp.where(g_idx < n_active, be, be[n_active - 1]).astype(jnp.int32)

    out = pl.pallas_call(
        _moe_body,
        grid_spec=pltpu.PrefetchScalarGridSpec(
            num_scalar_prefetch=2,
            grid=(G,),
            in_specs=[
                pl.BlockSpec((T, HIDDEN), lambda g, be_r, na_r: (0, 0)),
                pl.BlockSpec((1, T), lambda g, be_r, na_r: (0, 0)),
                pl.BlockSpec((1, T), lambda g, be_r, na_r: (0, 0)),
                pl.BlockSpec((1, T), lambda g, be_r, na_r: (0, 0)),
                pl.BlockSpec((1, T), lambda g, be_r, na_r: (0, 0)),
                pl.BlockSpec((1, HIDDEN, FF),
                             lambda g, be_r, na_r: (be_r[g], 0, 0)),
                pl.BlockSpec((1, 1, FF),
                             lambda g, be_r, na_r: (be_r[g], 0, 0)),
                pl.BlockSpec((1, FF, HIDDEN),
                             lambda g, be_r, na_r: (be_r[g], 0, 0)),
                pl.BlockSpec((1, 1, HIDDEN),
                             lambda g, be_r, na_r: (be_r[g], 0, 0)),
            ],
            out_specs=pl.BlockSpec((T, HIDDEN), lambda g, be_r, na_r: (0, 0)),
        ),
        out_shape=jax.ShapeDtypeStruct((T, HIDDEN), jnp.float32),
    )(be, jnp.reshape(n_active, (1,)), x_flat,
      dest0.reshape(1, T), dest1.reshape(1, T),
      gates[:, 0].reshape(1, T), gates[:, 1].reshape(1, T),
      W1, b1.reshape(E, 1, FF), W2, b2.reshape(E, 1, HIDDEN))
    return out.reshape(B, S, H)
